# Initial kernel scaffold; baseline (speedup 1.0000x reference)
import functools

import jax
import jax.numpy as jnp
from jax import lax
from jax.experimental import pallas as pl
from jax.experimental.pallas import tpu as pltpu
from jax.experimental.pallas import tpu_sc as plsc

N = 10000
E = 320000
D_IN = 128
D_H = 64
N_GRAPHS = 64

NC = 2
NS = 16
NW = NC * NS
K = 128
CHUNKS = E // K

_F32 = jnp.float32
_HIGH = lax.Precision.HIGHEST


def _dot(a, b):
    return lax.dot_general(a, b, (((1,), (0,)), ((), ())),
                           preferred_element_type=_F32, precision=_HIGH)


def _dot_t(a, b):
    return lax.dot_general(a, b, (((0,), (0,)), ((), ())),
                           preferred_element_type=_F32, precision=_HIGH)


def _sc_agg_body(feat_hbm, src_hbm, dst_hbm, zeros_hbm, out_hbm,
                 sidx_v, didx_v, rows_v, acc_sh):
    cid = lax.axis_index("c")
    sid = lax.axis_index("s")
    rows_per_sub = N // NS

    pltpu.sync_copy(zeros_hbm.at[pl.ds(sid * rows_per_sub, rows_per_sub)],
                    acc_sh.at[pl.ds(sid * rows_per_sub, rows_per_sub)])
    plsc.subcore_barrier()

    wid = cid * NS + sid

    @pl.loop(wid, CHUNKS, step=NW)
    def _(chunk):
        base = chunk * K
        pltpu.sync_copy(src_hbm.at[pl.ds(base, K)], sidx_v)
        pltpu.sync_copy(dst_hbm.at[pl.ds(base, K)], didx_v)
        pltpu.sync_copy(feat_hbm.at[sidx_v], rows_v)
        pltpu.sync_copy(rows_v, acc_sh.at[didx_v], add=True)

    plsc.subcore_barrier()
    pltpu.sync_copy(acc_sh.at[pl.ds(sid * rows_per_sub, rows_per_sub)],
                    out_hbm.at[cid, pl.ds(sid * rows_per_sub, rows_per_sub)])


_sc_agg = pl.kernel(
    _sc_agg_body,
    out_type=jax.ShapeDtypeStruct((NC, N, D_H), _F32),
    mesh=plsc.VectorSubcoreMesh(core_axis_name="c", subcore_axis_name="s"),
    scratch_types=[
        pltpu.VMEM((K,), jnp.int32),
        pltpu.VMEM((K,), jnp.int32),
        pltpu.VMEM((K, D_H), _F32),
        pltpu.VMEM_SHARED((N, D_H), _F32),
    ],
)


def _tc_pre_body(x_ref, wrel_ref, wroot_ref, xr_out, root_out):
    x = x_ref[...]
    xr_out[...] = _dot(x, wrel_ref[...])
    root_out[...] = _dot(x, wroot_ref[...])


def _tc_pre(x, wrel, wroot):
    return pl.pallas_call(
        _tc_pre_body,
        out_shape=(jax.ShapeDtypeStruct((N, D_H), _F32),
                   jax.ShapeDtypeStruct((N, D_H), _F32)),
    )(x, wrel, wroot)


def _tc_mid_body(aggp_ref, root_ref, b_ref, gamma_ref, beta_ref,
                 wrel_ref, wroot_ref, xr_out, root_out):
    pre = aggp_ref[0] + aggp_ref[1] + root_ref[...] + b_ref[...]
    mu = jnp.mean(pre, axis=0, keepdims=True)
    var = jnp.mean((pre - mu) * (pre - mu), axis=0, keepdims=True)
    h = gamma_ref[...] * (pre - mu) * lax.rsqrt(var + 1e-5) + beta_ref[...]
    h = jnp.maximum(h, 0.0)
    xr_out[...] = _dot(h, wrel_ref[...])
    root_out[...] = _dot(h, wroot_ref[...])


def _tc_mid(aggp, root, b, gamma, beta, wrel, wroot):
    return pl.pallas_call(
        _tc_mid_body,
        out_shape=(jax.ShapeDtypeStruct((N, D_H), _F32),
                   jax.ShapeDtypeStruct((N, D_H), _F32)),
    )(aggp, root, b.reshape(1, D_H), gamma.reshape(1, D_H),
      beta.reshape(1, D_H), wrel, wroot)


def _tc_final_body(aggp_ref, root_ref, b_ref, batch_ref, wc1_ref, bc1_ref,
                   wc2_ref, bc2_ref, out_ref):
    h = jnp.maximum(aggp_ref[0] + aggp_ref[1] + root_ref[...] + b_ref[...], 0.0)
    seg = batch_ref[...]
    ids = lax.broadcasted_iota(jnp.int32, (1, N_GRAPHS), 1)
    mask = (seg == ids).astype(_F32)
    s = _dot_t(mask, h)
    cnt = _dot_t(mask, jnp.ones((N, 1), _F32))
    g = s / jnp.maximum(cnt, 1.0)
    g = jnp.maximum(_dot(g, wc1_ref[...]) + bc1_ref[...], 0.0)
    out_ref[...] = _dot(g, wc2_ref[...]) + bc2_ref[...]


def _tc_final(aggp, root, b, batch, wc1, bc1, wc2, bc2):
    return pl.pallas_call(
        _tc_final_body,
        out_shape=jax.ShapeDtypeStruct((N_GRAPHS, 1), _F32),
    )(aggp, root, b.reshape(1, D_H), batch.reshape(N, 1), wc1,
      bc1.reshape(1, D_H), wc2, bc2.reshape(1, 1))


def kernel(x, edge_index, batch, W_rel1, b_rel1, W_root1, gamma1, beta1,
           W_rel2, b_rel2, W_root2, gamma2, beta2,
           W_rel3, b_rel3, W_root3, Wc1, bc1, Wc2, bc2):
    src = edge_index[0]
    dst = edge_index[1]
    zeros = jnp.zeros((N, D_H), _F32)

    xr1, root1 = _tc_pre(x, W_rel1, W_root1)
    agg1 = _sc_agg(xr1, src, dst, zeros)
    xr2, root2 = _tc_mid(agg1, root1, b_rel1, gamma1, beta1, W_rel2, W_root2)
    agg2 = _sc_agg(xr2, src, dst, zeros)
    xr3, root3 = _tc_mid(agg2, root2, b_rel2, gamma2, beta2, W_rel3, W_root3)
    agg3 = _sc_agg(xr3, src, dst, zeros)
    return _tc_final(agg3, root3, b_rel3, batch, Wc1, bc1, Wc2, bc2)


# speedup vs baseline: 6.8908x; 6.8908x over previous
import functools

import jax
import jax.numpy as jnp
from jax import lax
from jax.experimental import pallas as pl
from jax.experimental.pallas import tpu as pltpu
from jax.experimental.pallas import tpu_sc as plsc

N = 10000
E = 320000
D_IN = 128
D_H = 64
N_GRAPHS = 64

NC = 2
NS = 16
NW = NC * NS
K = 128
CHUNKS = E // K

_F32 = jnp.float32
_HIGH = lax.Precision.HIGHEST


def _dot(a, b):
    return lax.dot_general(a, b, (((1,), (0,)), ((), ())),
                           preferred_element_type=_F32, precision=_HIGH)


def _dot_t(a, b):
    return lax.dot_general(a, b, (((0,), (0,)), ((), ())),
                           preferred_element_type=_F32, precision=_HIGH)


_ROWS_PER_SUB = 624
_TAIL = N - NS * _ROWS_PER_SUB


def _sc_agg_body(feat_hbm, src_hbm, dst_hbm, zeros_hbm, out_hbm,
                 sidx_v, didx_v, rows_v, acc_sh):
    cid = lax.axis_index("c")
    sid = lax.axis_index("s")

    pltpu.sync_copy(zeros_hbm.at[pl.ds(sid * _ROWS_PER_SUB, _ROWS_PER_SUB)],
                    acc_sh.at[pl.ds(sid * _ROWS_PER_SUB, _ROWS_PER_SUB)])

    @pl.when(sid == NS - 1)
    def _():
        pltpu.sync_copy(zeros_hbm.at[pl.ds(NS * _ROWS_PER_SUB, _TAIL)],
                        acc_sh.at[pl.ds(NS * _ROWS_PER_SUB, _TAIL)])

    plsc.subcore_barrier()

    wid = cid * NS + sid

    @pl.loop(wid, CHUNKS, step=NW)
    def _(chunk):
        base = chunk * K
        pltpu.sync_copy(src_hbm.at[pl.ds(base, K)], sidx_v)
        pltpu.sync_copy(dst_hbm.at[pl.ds(base, K)], didx_v)
        pltpu.sync_copy(feat_hbm.at[sidx_v], rows_v)
        pltpu.sync_copy(rows_v, acc_sh.at[didx_v], add=True)

    plsc.subcore_barrier()
    pltpu.sync_copy(acc_sh.at[pl.ds(sid * _ROWS_PER_SUB, _ROWS_PER_SUB)],
                    out_hbm.at[cid, pl.ds(sid * _ROWS_PER_SUB, _ROWS_PER_SUB)])

    @pl.when(sid == NS - 1)
    def _():
        pltpu.sync_copy(acc_sh.at[pl.ds(NS * _ROWS_PER_SUB, _TAIL)],
                        out_hbm.at[cid, pl.ds(NS * _ROWS_PER_SUB, _TAIL)])


_sc_agg = pl.kernel(
    _sc_agg_body,
    out_type=jax.ShapeDtypeStruct((NC, N, D_H), _F32),
    mesh=plsc.VectorSubcoreMesh(core_axis_name="c", subcore_axis_name="s"),
    scratch_types=[
        pltpu.VMEM((K,), jnp.int32),
        pltpu.VMEM((K,), jnp.int32),
        pltpu.VMEM((K, D_H), _F32),
        pltpu.VMEM_SHARED((N, D_H), _F32),
    ],
    compiler_params=pltpu.CompilerParams(use_tc_tiling_on_sc=False),
)


def _tc_pre_body(x_ref, wrel_ref, wroot_ref, xr_out, root_out):
    x = x_ref[...]
    xr_out[...] = _dot(x, wrel_ref[...])
    root_out[...] = _dot(x, wroot_ref[...])


def _tc_pre(x, wrel, wroot):
    return pl.pallas_call(
        _tc_pre_body,
        out_shape=(jax.ShapeDtypeStruct((N, D_H), _F32),
                   jax.ShapeDtypeStruct((N, D_H), _F32)),
    )(x, wrel, wroot)


def _tc_mid_body(aggp_ref, root_ref, b_ref, gamma_ref, beta_ref,
                 wrel_ref, wroot_ref, xr_out, root_out):
    pre = aggp_ref[0] + aggp_ref[1] + root_ref[...] + b_ref[...]
    mu = jnp.mean(pre, axis=0, keepdims=True)
    var = jnp.mean((pre - mu) * (pre - mu), axis=0, keepdims=True)
    h = gamma_ref[...] * (pre - mu) * lax.rsqrt(var + 1e-5) + beta_ref[...]
    h = jnp.maximum(h, 0.0)
    xr_out[...] = _dot(h, wrel_ref[...])
    root_out[...] = _dot(h, wroot_ref[...])


def _tc_mid(aggp, root, b, gamma, beta, wrel, wroot):
    return pl.pallas_call(
        _tc_mid_body,
        out_shape=(jax.ShapeDtypeStruct((N, D_H), _F32),
                   jax.ShapeDtypeStruct((N, D_H), _F32)),
    )(aggp, root, b.reshape(1, D_H), gamma.reshape(1, D_H),
      beta.reshape(1, D_H), wrel, wroot)


def _tc_final_body(aggp_ref, root_ref, b_ref, batch_ref, wc1_ref, bc1_ref,
                   wc2_ref, bc2_ref, out_ref):
    h = jnp.maximum(aggp_ref[0] + aggp_ref[1] + root_ref[...] + b_ref[...], 0.0)
    seg = batch_ref[...]
    ids = lax.broadcasted_iota(jnp.int32, (1, N_GRAPHS), 1)
    mask = (seg == ids).astype(_F32)
    s = _dot_t(mask, h)
    cnt = _dot_t(mask, jnp.ones((N, 1), _F32))
    g = s / jnp.maximum(cnt, 1.0)
    g = jnp.maximum(_dot(g, wc1_ref[...]) + bc1_ref[...], 0.0)
    out_ref[...] = _dot(g, wc2_ref[...]) + bc2_ref[...]


def _tc_final(aggp, root, b, batch, wc1, bc1, wc2, bc2):
    return pl.pallas_call(
        _tc_final_body,
        out_shape=jax.ShapeDtypeStruct((N_GRAPHS, 1), _F32),
    )(aggp, root, b.reshape(1, D_H), batch.reshape(N, 1), wc1,
      bc1.reshape(1, D_H), wc2, bc2.reshape(1, 1))


def kernel(x, edge_index, batch, W_rel1, b_rel1, W_root1, gamma1, beta1,
           W_rel2, b_rel2, W_root2, gamma2, beta2,
           W_rel3, b_rel3, W_root3, Wc1, bc1, Wc2, bc2):
    src = edge_index[0]
    dst = edge_index[1]
    zeros = jnp.zeros((N, D_H), _F32)

    xr1, root1 = _tc_pre(x, W_rel1, W_root1)
    agg1 = _sc_agg(xr1, src, dst, zeros)
    xr2, root2 = _tc_mid(agg1, root1, b_rel1, gamma1, beta1, W_rel2, W_root2)
    agg2 = _sc_agg(xr2, src, dst, zeros)
    xr3, root3 = _tc_mid(agg2, root2, b_rel2, gamma2, beta2, W_rel3, W_root3)
    agg3 = _sc_agg(xr3, src, dst, zeros)
    return _tc_final(agg3, root3, b_rel3, batch, Wc1, bc1, Wc2, bc2)
